# baseline (speedup 1.0000x reference)
import functools

import jax
import jax.numpy as jnp
from jax import lax
from jax.experimental import pallas as pl
from jax.experimental.pallas import tpu as pltpu
from jax.experimental.pallas import tpu_sc as plsc

B = 2
C = 12
J = 23
JP = 32
H, W = 128, 640
HW = H * W
G = 52
G3 = G * G * G
GRID_SPACING = 2.0
OFFSET = -100.0

NC, NS = 2, 16
NW = NC * NS
RPB = 128
BPT = 36
G3P = NW * BPT * RPB
INV_C = 1.0 / C

_mesh = plsc.VectorSubcoreMesh(core_axis_name="c", subcore_axis_name="s")


JW = JP // 2
_HI = -65536
_LO = 65535


@functools.partial(
    pl.kernel,
    out_type=jax.ShapeDtypeStruct((B, NW, BPT, RPB, JW), jnp.int32),
    mesh=_mesh,
    scratch_types=[
        pltpu.VMEM((C, BPT, RPB), jnp.int32),
        pltpu.VMEM((2, C, RPB, JW), jnp.int32),
        pltpu.VMEM((2, RPB, JW), jnp.int32),
        pltpu.SemaphoreType.DMA,
        pltpu.SemaphoreType.DMA,
        pltpu.SemaphoreType.DMA,
    ],
    compiler_params=pltpu.CompilerParams(
        use_tc_tiling_on_sc=False, needs_layout_passes=False
    ),
)
def _sc_reproject(table_hbm, idx_hbm, out_hbm, idx_v, rows_v, out_v, sem0, sem1, sem_o):
    wid = lax.axis_index("s") * NC + lax.axis_index("c")
    sems = (sem0, sem1)

    def fire(blk, buf):
        for cc in range(C):
            pltpu.async_copy(
                table_hbm.at[idx_v.at[cc, blk]], rows_v.at[buf, cc], sems[buf]
            )

    def drain(buf):
        for cc in range(C):
            pltpu.make_async_copy(
                table_hbm.at[idx_v.at[cc, 0]], rows_v.at[buf, cc], sems[buf]
            ).wait()

    def compute(buf, par):
        def row_body(i, _):
            acc_e = jnp.zeros((JW,), jnp.float32)
            acc_o = jnp.zeros((JW,), jnp.float32)
            for cc in range(C):
                v = rows_v[buf, cc, i, :]
                acc_e = acc_e + plsc.bitcast(v << 16, jnp.float32)
                acc_o = acc_o + plsc.bitcast(v & _HI, jnp.float32)
            we = plsc.bitcast(acc_e * INV_C, jnp.int32)
            wo = plsc.bitcast(acc_o * INV_C, jnp.int32)
            out_v[par, i, :] = ((we >> 16) & _LO) | (wo & _HI)
            return 0

        lax.fori_loop(0, RPB, row_body, 0)

    for b in range(B):
        pltpu.sync_copy(idx_hbm.at[b, :, wid], idx_v)
        fire(0, 0)

        def pair_body(blk2, _, b=b):
            for par in range(2):
                blk = 2 * blk2 + par
                drain(par)
                nxt = blk + 1

                @pl.when(nxt < BPT)
                def _():
                    fire(nxt, 1 - par)

                @pl.when(blk2 > 0)
                def _():
                    pltpu.make_async_copy(
                        out_hbm.at[b, 0, 0], out_v.at[par], sem_o
                    ).wait()

                compute(par, par)
                pltpu.async_copy(out_v.at[par], out_hbm.at[b, wid, blk], sem_o)
            return 0

        lax.fori_loop(0, BPT // 2, pair_body, 0)
        for par in range(2):
            pltpu.make_async_copy(out_hbm.at[b, 0, 0], out_v.at[par], sem_o).wait()


def kernel(heatmaps, center, reproLookup):
    ci = ((center - OFFSET) / GRID_SPACING).astype(jnp.int32)
    crops = []
    for b in range(B):
        start = (jnp.int32(0), ci[b, 0] - G // 2, ci[b, 1] - G // 2, ci[b, 2] - G // 2)
        crops.append(lax.dynamic_slice(reproLookup, start, (C, G, G, G)))
    idx = jnp.stack(crops).reshape(B, C, G3)
    offs = (jnp.arange(B, dtype=jnp.int32)[:, None] * C
            + jnp.arange(C, dtype=jnp.int32)[None, :]) * HW
    idx = idx + offs[:, :, None]
    idx = jnp.pad(idx, ((0, 0), (0, 0), (0, G3P - G3)))
    idx = idx.reshape(B, C, NW, BPT, RPB)
    hm_t = jnp.transpose(heatmaps.reshape(B, C, J, HW), (0, 1, 3, 2))
    hm_t = jnp.pad(hm_t, ((0, 0), (0, 0), (0, 0), (0, JP - J)))
    table = hm_t.reshape(B * C * HW, JW, 2).astype(jnp.bfloat16)
    table = lax.bitcast_convert_type(table, jnp.int32)

    out = _sc_reproject(table, idx)
    out = lax.bitcast_convert_type(out, jnp.bfloat16)
    out = out.reshape(B, G3P, JP)[:, :G3, :J].astype(jnp.float32)
    return jnp.transpose(out, (0, 2, 1)).reshape(B, J, G, G, G)


# speedup vs baseline: 1.0454x; 1.0454x over previous
import functools

import jax
import jax.numpy as jnp
from jax import lax
from jax.experimental import pallas as pl
from jax.experimental.pallas import tpu as pltpu
from jax.experimental.pallas import tpu_sc as plsc

B = 2
C = 12
J = 23
JP = 32
H, W = 128, 640
HW = H * W
G = 52
G3 = G * G * G
GRID_SPACING = 2.0
OFFSET = -100.0

NC, NS = 2, 16
NW = NC * NS
RPB = 128
BPT = 35
G3P = NW * BPT * RPB
INV_C = 1.0 / C

_mesh = plsc.VectorSubcoreMesh(core_axis_name="c", subcore_axis_name="s")


@functools.partial(
    pl.kernel,
    out_type=jax.ShapeDtypeStruct((B, G3P, JP), jnp.float32),
    mesh=_mesh,
    scratch_types=[
        pltpu.VMEM((C, RPB), jnp.int32),
        pltpu.VMEM((C, RPB, JP), jnp.float32),
        pltpu.VMEM((RPB, JP), jnp.float32),
        pltpu.SemaphoreType.DMA,
    ],
    compiler_params=pltpu.CompilerParams(use_tc_tiling_on_sc=False),
)
def _sc_reproject(table_hbm, idx_hbm, out_hbm, idx_v, rows_v, out_v, sem):
    wid = lax.axis_index("s") * NC + lax.axis_index("c")

    for b in range(B):
        def blk_body(blk, _, b=b):
            base = (wid * BPT + blk) * RPB
            pltpu.sync_copy(idx_hbm.at[b, :, pl.ds(base, RPB)], idx_v)
            copies = [
                pltpu.async_copy(table_hbm.at[idx_v.at[cc]], rows_v.at[cc], sem)
                for cc in range(C)
            ]
            for cp in copies:
                cp.wait()

            def row_body(i, _):
                for h in range(2):
                    sl = pl.ds(h * 16, 16)
                    acc = rows_v[0, i, sl]
                    for cc in range(1, C):
                        acc = acc + rows_v[cc, i, sl]
                    out_v[i, sl] = acc * INV_C
                return 0

            lax.fori_loop(0, RPB, row_body, 0)
            pltpu.sync_copy(out_v, out_hbm.at[b, pl.ds(base, RPB), :])
            return 0

        lax.fori_loop(0, BPT, blk_body, 0)


TBS = 2048


def _pack_block(hm_ref, out_ref):
    x = hm_ref[0]
    xp = jnp.concatenate([x, jnp.zeros((JP - J, TBS), x.dtype)], axis=0)
    out_ref[...] = xp.T


def _build_table(heatmaps):
    hm = heatmaps.reshape(B * C, J, HW)
    return pl.pallas_call(
        _pack_block,
        out_shape=jax.ShapeDtypeStruct((B * C * HW, JP), jnp.float32),
        grid=(B * C, HW // TBS),
        in_specs=[pl.BlockSpec((1, J, TBS), lambda bc, k: (bc, 0, k))],
        out_specs=pl.BlockSpec((TBS, JP), lambda bc, k: (bc * (HW // TBS) + k, 0)),
    )(hm)


def kernel(heatmaps, center, reproLookup):
    ci = ((center - OFFSET) / GRID_SPACING).astype(jnp.int32)
    crops = []
    for b in range(B):
        start = (jnp.int32(0), ci[b, 0] - G // 2, ci[b, 1] - G // 2, ci[b, 2] - G // 2)
        crops.append(lax.dynamic_slice(reproLookup, start, (C, G, G, G)))
    idx = jnp.stack(crops).reshape(B, C, G3)
    offs = (jnp.arange(B, dtype=jnp.int32)[:, None] * C
            + jnp.arange(C, dtype=jnp.int32)[None, :]) * HW
    idx = idx + offs[:, :, None]
    idx = jnp.pad(idx, ((0, 0), (0, 0), (0, G3P - G3)))
    table = _build_table(heatmaps)

    out = _sc_reproject(table, idx)
    out = out[:, :G3, :J]
    return jnp.transpose(out, (0, 2, 1)).reshape(B, J, G, G, G)


# speedup vs baseline: 1.4820x; 1.4177x over previous
import functools

import jax
import jax.numpy as jnp
from jax import lax
from jax.experimental import pallas as pl
from jax.experimental.pallas import tpu as pltpu
from jax.experimental.pallas import tpu_sc as plsc

B = 2
C = 12
J = 23
JP = 32
H, W = 128, 640
HW = H * W
G = 52
G3 = G * G * G
GRID_SPACING = 2.0
OFFSET = -100.0

NC, NS = 2, 16
NW = NC * NS
RPB = 128
BPT = 35
G3P = NW * BPT * RPB
INV_C = 1.0 / C

_mesh = plsc.VectorSubcoreMesh(core_axis_name="c", subcore_axis_name="s")


@functools.partial(
    pl.kernel,
    out_type=jax.ShapeDtypeStruct((B, G3P, JP), jnp.float32),
    mesh=_mesh,
    scratch_types=[
        pltpu.VMEM((C, RPB), jnp.int32),
        pltpu.VMEM((C, RPB, JP), jnp.float32),
        pltpu.VMEM((RPB, JP), jnp.float32),
        pltpu.SemaphoreType.DMA,
    ],
    compiler_params=pltpu.CompilerParams(use_tc_tiling_on_sc=False),
)
def _sc_reproject(table_hbm, idx_hbm, out_hbm, idx_v, rows_v, out_v, sem):
    wid = lax.axis_index("s") * NC + lax.axis_index("c")

    for b in range(B):
        def blk_body(blk, _, b=b):
            base = (wid * BPT + blk) * RPB
            pltpu.sync_copy(idx_hbm.at[b, :, pl.ds(base, RPB)], idx_v)
            copies = [
                pltpu.async_copy(table_hbm.at[idx_v.at[cc]], rows_v.at[cc], sem)
                for cc in range(C)
            ]
            for cp in copies:
                cp.wait()

            def row_body(i, _):
                for h in range(2):
                    sl = pl.ds(h * 16, 16)
                    acc = rows_v[0, i, sl]
                    for cc in range(1, C):
                        acc = acc + rows_v[cc, i, sl]
                    out_v[i, sl] = acc * INV_C
                return 0

            lax.fori_loop(0, RPB, row_body, 0)
            pltpu.sync_copy(out_v, out_hbm.at[b, pl.ds(base, RPB), :])
            return 0

        lax.fori_loop(0, BPT, blk_body, 0)


TBS = 4096


def _pack_block(hm_ref, out_ref):
    x = hm_ref[0]
    xp = jnp.concatenate([x, jnp.zeros((JP - J, TBS), x.dtype)], axis=0)
    q = TBS // 4
    out_ref[...] = jnp.concatenate(
        [xp[:, i * q:(i + 1) * q].T for i in range(4)], axis=1
    )


def _build_table(heatmaps):
    hm = heatmaps.reshape(B * C, J, HW)
    packed = pl.pallas_call(
        _pack_block,
        out_shape=jax.ShapeDtypeStruct((B * C * HW // 4, 4 * JP), jnp.float32),
        grid=(B * C, HW // TBS),
        in_specs=[pl.BlockSpec((1, J, TBS), lambda bc, k: (bc, 0, k))],
        out_specs=pl.BlockSpec(
            (TBS // 4, 4 * JP), lambda bc, k: (bc * (HW // TBS) + k, 0)
        ),
    )(hm)
    return packed.reshape(B * C * HW, JP)


def kernel(heatmaps, center, reproLookup):
    ci = ((center - OFFSET) / GRID_SPACING).astype(jnp.int32)
    crops = []
    for b in range(B):
        start = (jnp.int32(0), ci[b, 0] - G // 2, ci[b, 1] - G // 2, ci[b, 2] - G // 2)
        crops.append(lax.dynamic_slice(reproLookup, start, (C, G, G, G)))
    idx = jnp.stack(crops).reshape(B, C, G3)
    offs = (jnp.arange(B, dtype=jnp.int32)[:, None] * C
            + jnp.arange(C, dtype=jnp.int32)[None, :]) * HW
    t = idx & (TBS - 1)
    idx = (idx & ~(TBS - 1)) + ((t & (TBS // 4 - 1)) << 2) + (t >> 10)
    idx = idx + offs[:, :, None]
    idx = jnp.pad(idx, ((0, 0), (0, 0), (0, G3P - G3)))
    table = _build_table(heatmaps)

    out = _sc_reproject(table, idx)
    out = out[:, :G3, :J]
    return jnp.transpose(out, (0, 2, 1)).reshape(B, J, G, G, G)
